# Initial kernel scaffold; baseline (speedup 1.0000x reference)
import jax
import jax.numpy as jnp
from jax.experimental import pallas as pl


def kernel(dense_points, sparse_points, normalized_scores, approach_directions, normalized_view_score, template_views):
    raise NotImplementedError("write your pallas kernel here")


# speedup vs baseline: 23.0097x; 23.0097x over previous
import jax
import jax.numpy as jnp
from jax.experimental import pallas as pl


def _aff_body(dn_ref, sp_ref, sc_ref, out_ref):
    dn = dn_ref[0, 0]
    sp = sp_ref[0]
    sc = sc_ref[0]
    M = sp.shape[0]
    aa = jnp.sum(dn * dn, axis=0, keepdims=True)
    bb = jnp.sum(sp * sp, axis=1, keepdims=True)
    ab = jax.lax.dot_general(sp, dn, (((1,), (0,)), ((), ())),
                             preferred_element_type=jnp.float32)
    d2 = (aa + bb) - 2.0 * ab
    iota = jax.lax.broadcasted_iota(jnp.int32, d2.shape, 0)
    m1 = jnp.min(d2, axis=0, keepdims=True)
    i1 = jnp.min(jnp.where(d2 == m1, iota, M), axis=0, keepdims=True)
    hit1 = iota == i1
    s1 = jnp.max(jnp.where(hit1, sc, -1.0), axis=0, keepdims=True)
    d2b = jnp.where(hit1, jnp.inf, d2)
    m2 = jnp.min(d2b, axis=0, keepdims=True)
    i2 = jnp.min(jnp.where(d2b == m2, iota, M), axis=0, keepdims=True)
    s2 = jnp.max(jnp.where(iota == i2, sc, -1.0), axis=0, keepdims=True)
    out_ref[0, 0] = (s1 + s2) * 0.5


def _vs_body(ad_ref, tv_ref, nvs_ref, out_ref):
    tv = tv_ref[...]
    V = tv.shape[1]
    bbv = jnp.sum(tv * tv, axis=0, keepdims=True)
    nvs = nvs_ref[0]
    M = nvs.shape[0]
    iota = jax.lax.broadcasted_iota(jnp.int32, (M, V), 1)
    vs = jnp.zeros((M, V), jnp.float32)
    for j in range(3):
        dj = ad_ref[0, j]
        aaj = jnp.sum(dj * dj, axis=1, keepdims=True)
        dots = jax.lax.dot_general(dj, tv, (((1,), (0,)), ((), ())),
                                   preferred_element_type=jnp.float32)
        dv = jnp.sqrt(jnp.maximum((aaj + bbv) - 2.0 * dots, 0.0))
        mv = jnp.min(dv, axis=1, keepdims=True)
        vi = jnp.min(jnp.where(dv == mv, iota, V), axis=1, keepdims=True)
        vs = jnp.where(iota == vi, nvs[:, j:j + 1], vs)
    out_ref[0] = vs


def kernel(dense_points, sparse_points, normalized_scores,
           approach_directions, normalized_view_score, template_views):
    B, N, _ = dense_points.shape
    M = sparse_points.shape[1]
    V = template_views.shape[0]
    NT = 10
    TN = N // NT

    dnT = dense_points.reshape(B, NT, TN, 3).transpose(0, 1, 3, 2)
    sc2 = normalized_scores[:, :, None]
    adT = approach_directions.transpose(0, 2, 1, 3)
    tvT = template_views.T

    aff4 = pl.pallas_call(
        _aff_body,
        grid=(B, NT),
        in_specs=[
            pl.BlockSpec((1, 1, 3, TN), lambda b, t: (b, t, 0, 0)),
            pl.BlockSpec((1, M, 3), lambda b, t: (b, 0, 0)),
            pl.BlockSpec((1, M, 1), lambda b, t: (b, 0, 0)),
        ],
        out_specs=pl.BlockSpec((1, 1, 1, TN), lambda b, t: (b, t, 0, 0)),
        out_shape=jax.ShapeDtypeStruct((B, NT, 1, TN), jnp.float32),
    )(dnT, sparse_points, sc2)
    aff = aff4.reshape(B, N)

    vs = pl.pallas_call(
        _vs_body,
        grid=(B,),
        in_specs=[
            pl.BlockSpec((1, 3, M, 3), lambda b: (b, 0, 0, 0)),
            pl.BlockSpec((3, V), lambda b: (0, 0)),
            pl.BlockSpec((1, M, 3), lambda b: (b, 0, 0)),
        ],
        out_specs=pl.BlockSpec((1, M, V), lambda b: (b, 0, 0)),
        out_shape=jax.ShapeDtypeStruct((B, M, V), jnp.float32),
    )(adT, tvT, normalized_view_score)

    return aff, vs


# speedup vs baseline: 30.1480x; 1.3102x over previous
import jax
import jax.numpy as jnp
from jax.experimental import pallas as pl


def _aff_body(dn_ref, sp_ref, sc_ref, out_ref):
    dn = dn_ref[0, 0]
    sp = sp_ref[0]
    sc = sc_ref[0]
    M = sp.shape[0]
    aa = jnp.sum(dn * dn, axis=0, keepdims=True)
    bb = jnp.sum(sp * sp, axis=1, keepdims=True)
    ab = jax.lax.dot_general(sp, dn, (((1,), (0,)), ((), ())),
                             preferred_element_type=jnp.float32)
    d2 = (aa + bb) - 2.0 * ab
    m1 = jnp.min(d2, axis=0, keepdims=True)
    eq1 = d2 == m1
    s1 = jnp.max(jnp.where(eq1, sc, -1.0), axis=0, keepdims=True)
    d2b = jnp.where(eq1, jnp.inf, d2)
    m2 = jnp.min(d2b, axis=0, keepdims=True)
    s2 = jnp.max(jnp.where(d2b == m2, sc, -1.0), axis=0, keepdims=True)
    out_ref[0, 0] = (s1 + s2) * 0.5


def _vs_body(ad_ref, tv_ref, nvs_ref, out_ref):
    tv = tv_ref[...]
    V = tv.shape[1]
    bbv = jnp.sum(tv * tv, axis=0, keepdims=True)
    nvs = nvs_ref[0]
    M = nvs.shape[0]
    iota = jax.lax.broadcasted_iota(jnp.int32, (M, V), 1)
    vs = jnp.zeros((M, V), jnp.float32)
    for j in range(3):
        dj = ad_ref[0, j]
        aaj = jnp.sum(dj * dj, axis=1, keepdims=True)
        dots = jax.lax.dot_general(dj, tv, (((1,), (0,)), ((), ())),
                                   preferred_element_type=jnp.float32)
        dv = jnp.sqrt(jnp.maximum((aaj + bbv) - 2.0 * dots, 0.0))
        mv = jnp.min(dv, axis=1, keepdims=True)
        vi = jnp.min(jnp.where(dv == mv, iota, V), axis=1, keepdims=True)
        vs = jnp.where(iota == vi, nvs[:, j:j + 1], vs)
    out_ref[0] = vs


def kernel(dense_points, sparse_points, normalized_scores,
           approach_directions, normalized_view_score, template_views):
    B, N, _ = dense_points.shape
    M = sparse_points.shape[1]
    V = template_views.shape[0]
    NT = 10
    TN = N // NT

    dnT = dense_points.reshape(B, NT, TN, 3).transpose(0, 1, 3, 2)
    sc2 = normalized_scores[:, :, None]
    adT = approach_directions.transpose(0, 2, 1, 3)
    tvT = template_views.T

    aff4 = pl.pallas_call(
        _aff_body,
        grid=(B, NT),
        in_specs=[
            pl.BlockSpec((1, 1, 3, TN), lambda b, t: (b, t, 0, 0)),
            pl.BlockSpec((1, M, 3), lambda b, t: (b, 0, 0)),
            pl.BlockSpec((1, M, 1), lambda b, t: (b, 0, 0)),
        ],
        out_specs=pl.BlockSpec((1, 1, 1, TN), lambda b, t: (b, t, 0, 0)),
        out_shape=jax.ShapeDtypeStruct((B, NT, 1, TN), jnp.float32),
    )(dnT, sparse_points, sc2)
    aff = aff4.reshape(B, N)

    vs = pl.pallas_call(
        _vs_body,
        grid=(B,),
        in_specs=[
            pl.BlockSpec((1, 3, M, 3), lambda b: (b, 0, 0, 0)),
            pl.BlockSpec((3, V), lambda b: (0, 0)),
            pl.BlockSpec((1, M, 3), lambda b: (b, 0, 0)),
        ],
        out_specs=pl.BlockSpec((1, M, V), lambda b: (b, 0, 0)),
        out_shape=jax.ShapeDtypeStruct((B, M, V), jnp.float32),
    )(adT, tvT, normalized_view_score)

    return aff, vs
